# Initial kernel scaffold; baseline (speedup 1.0000x reference)
import jax
import jax.numpy as jnp
from jax.experimental import pallas as pl


def kernel(features, labels, centers, rule_matrix):
    raise NotImplementedError("write your pallas kernel here")


# speedup vs baseline: 9.7056x; 9.7056x over previous
import jax
import jax.numpy as jnp
from jax.experimental import pallas as pl
from jax.experimental.pallas import tpu as pltpu

NUM_CLASSES = 26
FEATURE_DIM = 128
BATCH = 16384
BLK = 4096
GRID = BATCH // BLK


def _body(feat_ref, lab_ref, cen_ref, rule_ref, out_ref, acc_ref):
    i = pl.program_id(0)

    cen = cen_ref[...]
    norms = jnp.sqrt(jnp.sum(cen * cen, axis=1, keepdims=True))
    nc = cen / jnp.maximum(norms, 1e-12)

    f = feat_ref[...]
    m = jax.lax.dot_general(f, nc, (((1,), (1,)), ((), ())),
                            preferred_element_type=jnp.float32)
    labs = lab_ref[0, 0, :]
    class_ids = jax.lax.broadcasted_iota(jnp.int32, (BLK, NUM_CLASSES), 1)
    sel = jnp.where(labs[:, None] == class_ids, m, 0.0)
    part = jnp.sum(sel)

    @pl.when(i == 0)
    def _():
        acc_ref[0, 0] = 0.0

    acc_ref[0, 0] += part

    @pl.when(i == GRID - 1)
    def _():
        n = NUM_CLASSES
        r0 = jax.lax.broadcasted_iota(jnp.int32, (n, n), 0)
        r1 = jax.lax.broadcasted_iota(jnp.int32, (n, n), 1)
        sim_w = jnp.where(r0 == r1, 0.0, rule_ref[...])
        wsum = jnp.sum(sim_w, axis=1, keepdims=True) + 1e-8
        nw = sim_w / wsum
        expected = jax.lax.dot_general(nw, nc, (((1,), (0,)), ((), ())),
                                       preferred_element_type=jnp.float32)
        cos_reg = jnp.sum(nc * expected, axis=1)
        loss_reg = 1.0 - jnp.sum(cos_reg) / n
        loss_center = 1.0 - acc_ref[0, 0] / BATCH
        out_ref[...] = jnp.reshape(loss_center + 0.5 * loss_reg, (1, 1))


def kernel(features, labels, centers, rule_matrix):
    labels3 = labels.astype(jnp.int32).reshape(GRID, 1, BLK)
    out = pl.pallas_call(
        _body,
        grid=(GRID,),
        in_specs=[
            pl.BlockSpec((BLK, FEATURE_DIM), lambda i: (i, 0)),
            pl.BlockSpec((1, 1, BLK), lambda i: (i, 0, 0)),
            pl.BlockSpec((NUM_CLASSES, FEATURE_DIM), lambda i: (0, 0)),
            pl.BlockSpec((NUM_CLASSES, NUM_CLASSES), lambda i: (0, 0)),
        ],
        out_specs=pl.BlockSpec((1, 1), lambda i: (0, 0)),
        out_shape=jax.ShapeDtypeStruct((1, 1), jnp.float32),
        scratch_shapes=[pltpu.SMEM((1, 1), jnp.float32)],
    )(features, labels3, centers, rule_matrix)
    return out[0, 0]
